# baseline (speedup 1.0000x reference)
import functools

import jax
import jax.numpy as jnp
from jax import lax
from jax.experimental import pallas as pl
from jax.experimental.pallas import tpu as pltpu
from jax.experimental.pallas import tpu_sc as plsc

BATCH = 16384
NCLS = 128
BBLK = 1024
NROWS = 100000
NWORK = 32
RPW = 3136
CROWS = 224
NCHUNK = RPW // CROWS
GPC = 74
IDXPAD = NCHUNK * GPC * 16
CPAD = RPW
WLPAD = 3200
ACH = 64


def _vgather(x, idx):
    dnums = lax.GatherDimensionNumbers(
        offset_dims=(), collapsed_slice_dims=(0,), start_index_map=(0,))
    return lax.gather(x, idx[:, None], dnums, (1,),
                      mode=lax.GatherScatterMode.PROMISE_IN_BOUNDS)


def _dense_body(out_ref, tgt_ref, wg_ref, wlg_ref, nwr_ref, l_ref):
    x = out_ref[...]
    m = jnp.max(x, axis=1, keepdims=True)
    lse = m + jnp.log(jnp.sum(jnp.exp(x - m), axis=1, keepdims=True))
    logp = x - lse
    part = jnp.sum(wg_ref[...] * tgt_ref[...] * logp)
    nwr_ref[...] = wlg_ref[...] * x

    @pl.when(pl.program_id(0) == 0)
    def _():
        l_ref[...] = jnp.zeros((1, 1), jnp.float32)

    l_ref[...] += jnp.full((1, 1), -part)


def _scatter_body(w_hbm, nwr_hbm, s_hbm, idx_hbm, out_hbm,
                  idx_v, claim_v, wloc_v, wb_v, sbuf_v, cbuf_v,
                  buf_v, obuf_v, sin0, sin1, sout0, sout1, gs0, gs1,
                  ss0, ss1):
    wid = lax.axis_index("c") * 16 + lax.axis_index("s")
    base = jnp.minimum(wid * RPW, NROWS - RPW)
    lane = lax.iota(jnp.int32, 16)

    pltpu.sync_copy(idx_hbm, idx_v.at[pl.ds(0, BATCH)])
    for j in range((IDXPAD - BATCH) // 16):
        idx_v[pl.ds(BATCH + j * 16, 16)] = jnp.full((16,), -1, jnp.int32)

    def _init_body(i, _):
        claim_v[pl.ds(i * 16, 16)] = jnp.full((16,), -1, jnp.int32)
        return 0

    lax.fori_loop(0, CPAD // 16, _init_body, 0)

    copy_h = pltpu.async_copy(w_hbm.at[pl.ds(base, RPW)],
                              out_hbm.at[pl.ds(base, RPW)], sin0)

    def _fire_in(t, cslot, sem):
        return pltpu.async_copy(w_hbm.at[pl.ds(base + t * CROWS, CROWS)],
                                cbuf_v.at[cslot], sem)

    def _out_dma(t, cslot, sem):
        return pltpu.make_async_copy(
            cbuf_v.at[cslot],
            out_hbm.at[pl.ds(base + t * CROWS, CROWS)], sem)

    def _claim_body(g, _):
        iv = idx_v[pl.ds(g * 16, 16)]
        local = iv - base
        inr = (local >= 0) & (local < RPW)
        bv = lane + g * 16
        plsc.store_scatter(claim_v, [local], bv, mask=inr)
        cb = plsc.load_gather(claim_v, [local], mask=inr)
        bad = inr & (cb != bv)
        anybad = plsc.all_reduce_population_count(bad)[0] > 0

        @pl.when(anybad)
        def _():
            win = inr
            for sh in range(1, 16):
                later = _vgather(local, jnp.minimum(lane + sh, 15))
                valid = (lane + sh) < 16
                win = win & ~(valid & (later == local))
            plsc.store_scatter(claim_v, [local], bv, mask=win)

        return 0

    lax.fori_loop(0, IDXPAD // 16, _claim_body, 0)
    copy_h.wait()

    def _compact_body(i, off):
        c = claim_v[pl.ds(i * 16, 16)]
        m = c >= 0
        pos = off + jnp.cumsum(m.astype(jnp.int32)) - 1
        plsc.store_scatter(wloc_v, [pos], lane + i * 16, mask=m)
        plsc.store_scatter(wb_v, [pos], c, mask=m)
        return off + plsc.all_reduce_population_count(m)[0]

    k_cnt = lax.fori_loop(0, CPAD // 16, _compact_body, jnp.int32(0))

    safe = jnp.maximum(k_cnt - 1, 0)
    rm = lax.rem(safe, 16)
    roff = safe - rm
    rms = jnp.broadcast_to(rm, (16,))
    lastloc = _vgather(wloc_v[pl.ds(roff, 16)], rms)
    lastb = _vgather(wb_v[pl.ds(roff, 16)], rms)
    for j in range(ACH // 16):
        plsc.store_scatter(wloc_v, [k_cnt + lane + j * 16], lastloc)
        plsc.store_scatter(wb_v, [k_cnt + lane + j * 16], lastb)

    plsc.subcore_barrier()
    ntrips = (k_cnt + ACH - 1) // ACH

    def _fire_gathers(t, aslot, sem):
        for j in range(ACH // 16):
            bv = wb_v[pl.ds(t * ACH + j * 16, 16)]
            pltpu.async_copy(nwr_hbm.at[bv],
                             buf_v.at[aslot, pl.ds(j * 16, 16)], sem)
            pltpu.async_copy(s_hbm.at[bv],
                             sbuf_v.at[aslot, pl.ds(j * 16, 16)], sem)

    def _wait_gathers(t, aslot, sem):
        for j in range(ACH // 16):
            bv = wb_v[pl.ds(t * ACH + j * 16, 16)]
            pltpu.make_async_copy(
                nwr_hbm.at[bv], buf_v.at[aslot, pl.ds(j * 16, 16)],
                sem).wait()
            pltpu.make_async_copy(
                s_hbm.at[bv], sbuf_v.at[aslot, pl.ds(j * 16, 16)],
                sem).wait()

    def _scatter_dmas(t, aslot, sem):
        hs = []
        for j in range(ACH // 16):
            lv = wloc_v[pl.ds(t * ACH + j * 16, 16)] + base
            hs.append(pltpu.make_async_copy(
                obuf_v.at[aslot, pl.ds(j * 16, 16)], out_hbm.at[lv], sem))
        return hs

    @pl.when(ntrips > 0)
    def _():
        _fire_gathers(0, 0, gs0)

    def _trip(t, _):
        def _one(aslot, gsem, ssem):
            @pl.when(t + 1 < ntrips)
            def _():
                _fire_gathers(t + 1, 1 - aslot, gs1 if aslot == 0 else gs0)

            _wait_gathers(t, aslot, gsem)

            @pl.when(t >= 2)
            def _():
                for h in _scatter_dmas(t - 2, aslot, ssem):
                    h.wait()

            def _rowdiv(r, _):
                rr = lax.rem(r, 16)
                sval = _vgather(sbuf_v[aslot, pl.ds(r - rr, 16)],
                                jnp.broadcast_to(rr, (16,)))
                inv = 1.0 / sval
                for c in range(NCLS // 16):
                    obuf_v[aslot, r, pl.ds(c * 16, 16)] = (
                        buf_v[aslot, r, pl.ds(c * 16, 16)] * inv)
                return 0

            lax.fori_loop(0, ACH, _rowdiv, 0)
            for h in _scatter_dmas(t, aslot, ssem):
                h.start()

        @pl.when(t % 2 == 0)
        def _():
            _one(0, gs0, ss0)

        @pl.when(t % 2 == 1)
        def _():
            _one(1, gs1, ss1)

        return 0

    lax.fori_loop(0, ntrips, _trip, 0)

    def _drain_scatters(sem):
        for j in range(ACH // 16):
            lv = wloc_v[pl.ds(j * 16, 16)] + base
            pltpu.make_async_copy(obuf_v.at[0, pl.ds(j * 16, 16)],
                                  out_hbm.at[lv], sem).wait()

    @pl.when(ntrips >= 2)
    def _():
        _drain_scatters(ss0)
        _drain_scatters(ss1)

    @pl.when(ntrips == 1)
    def _():
        _drain_scatters(ss0)


def _sc_scatter(weights, nw_raw, s, indices):
    mesh = plsc.VectorSubcoreMesh(core_axis_name="c", subcore_axis_name="s")
    f = functools.partial(
        pl.kernel,
        mesh=mesh,
        compiler_params=pltpu.CompilerParams(needs_layout_passes=False),
        out_type=jax.ShapeDtypeStruct((NROWS, NCLS), jnp.float32),
        scratch_types=[
            pltpu.VMEM((IDXPAD,), jnp.int32),
            pltpu.VMEM((CPAD,), jnp.int32),
            pltpu.VMEM((WLPAD,), jnp.int32),
            pltpu.VMEM((WLPAD,), jnp.int32),
            pltpu.VMEM((2, ACH), jnp.float32),
            pltpu.VMEM((2, CROWS, NCLS), jnp.float32),
            pltpu.VMEM((2, ACH, NCLS), jnp.float32),
            pltpu.VMEM((2, ACH, NCLS), jnp.float32),
            pltpu.SemaphoreType.DMA,
            pltpu.SemaphoreType.DMA,
            pltpu.SemaphoreType.DMA,
            pltpu.SemaphoreType.DMA,
            pltpu.SemaphoreType.DMA,
            pltpu.SemaphoreType.DMA,
            pltpu.SemaphoreType.DMA,
            pltpu.SemaphoreType.DMA,
        ],
    )(_scatter_body)
    return f(weights, nw_raw, s, indices)


def kernel(output, targets, weights, weak_labels, indices):
    w_g = jnp.take(weights, indices, axis=0)
    wl_g = jnp.take(weak_labels, indices, axis=0)
    grid = BATCH // BBLK
    row_spec = pl.BlockSpec((BBLK, NCLS), lambda i: (i, 0))
    nw_raw, lmat = pl.pallas_call(
        _dense_body,
        grid=(grid,),
        in_specs=[row_spec, row_spec, row_spec, row_spec],
        out_specs=[row_spec, pl.BlockSpec((1, 1), lambda i: (0, 0))],
        out_shape=[
            jax.ShapeDtypeStruct((BATCH, NCLS), jnp.float32),
            jax.ShapeDtypeStruct((1, 1), jnp.float32),
        ],
    )(output, targets, w_g, wl_g)
    s = jnp.sum(nw_raw, axis=1)
    updated = _sc_scatter(weights, nw_raw, s, indices)
    return lmat[0, 0], updated


# speedup vs baseline: 11.5206x; 11.5206x over previous
import functools

import jax
import jax.numpy as jnp
from jax import lax
from jax.experimental import pallas as pl
from jax.experimental.pallas import tpu as pltpu
from jax.experimental.pallas import tpu_sc as plsc

BATCH = 16384
NCLS = 128
BBLK = 1024
NROWS = 100000
NWORK = 32
RPW = 3136
CROWS = 224
NCHUNK = RPW // CROWS
GPC = 74
IDXPAD = NCHUNK * GPC * 16
CPAD = RPW
WLPAD = 3200
ACH = 64


def _vgather(x, idx):
    dnums = lax.GatherDimensionNumbers(
        offset_dims=(), collapsed_slice_dims=(0,), start_index_map=(0,))
    return lax.gather(x, idx[:, None], dnums, (1,),
                      mode=lax.GatherScatterMode.PROMISE_IN_BOUNDS)


def _dense_body(out_ref, tgt_ref, wg_ref, wlg_ref, nwr_ref, l_ref):
    x = out_ref[...]
    m = jnp.max(x, axis=1, keepdims=True)
    lse = m + jnp.log(jnp.sum(jnp.exp(x - m), axis=1, keepdims=True))
    logp = x - lse
    part = jnp.sum(wg_ref[...] * tgt_ref[...] * logp)
    nwr_ref[...] = wlg_ref[...] * x

    @pl.when(pl.program_id(0) == 0)
    def _():
        l_ref[...] = jnp.zeros((1, 1), jnp.float32)

    l_ref[...] += jnp.full((1, 1), -part)


def _claims_body(w_hbm, idx_hbm, out_hbm, wloc_hbm, wb_hbm, k_hbm,
                 idx_v, claim_v, wloc_v, wb_v, kbuf_v, cbuf_v,
                 sin0, sin1, sout0, sout1):
    wid = lax.axis_index("c") * 16 + lax.axis_index("s")
    base = jnp.minimum(wid * RPW, NROWS - RPW)
    lane = lax.iota(jnp.int32, 16)

    pltpu.sync_copy(idx_hbm, idx_v.at[pl.ds(0, BATCH)])
    for j in range((IDXPAD - BATCH) // 16):
        idx_v[pl.ds(BATCH + j * 16, 16)] = jnp.full((16,), -1, jnp.int32)

    def _init_body(i, _):
        claim_v[pl.ds(i * 16, 16)] = jnp.full((16,), -1, jnp.int32)
        return 0

    lax.fori_loop(0, CPAD // 16, _init_body, 0)

    def _fire_in(t, cslot, sem):
        return pltpu.async_copy(w_hbm.at[pl.ds(base + t * CROWS, CROWS)],
                                cbuf_v.at[cslot], sem)

    def _out_dma(t, cslot, sem):
        return pltpu.make_async_copy(
            cbuf_v.at[cslot],
            out_hbm.at[pl.ds(base + t * CROWS, CROWS)], sem)

    def _claim_body(g, _):
        iv = idx_v[pl.ds(g * 16, 16)]
        local = iv - base
        inr = (local >= 0) & (local < RPW)
        bv = lane + g * 16
        plsc.store_scatter(claim_v, [local], bv, mask=inr)
        cb = plsc.load_gather(claim_v, [local], mask=inr)
        bad = inr & (cb != bv)
        anybad = plsc.all_reduce_population_count(bad)[0] > 0

        @pl.when(anybad)
        def _():
            win = inr
            for sh in range(1, 16):
                later = _vgather(local, jnp.minimum(lane + sh, 15))
                valid = (lane + sh) < 16
                win = win & ~(valid & (later == local))
            plsc.store_scatter(claim_v, [local], bv, mask=win)

        return 0

    def _chunk_body(t, _):
        @pl.when(t % 2 == 0)
        def _():
            @pl.when(t >= 2)
            def _():
                _out_dma(t - 2, 0, sout0).wait()
            h = _fire_in(t, 0, sin0)
            lax.fori_loop(t * GPC, (t + 1) * GPC, _claim_body, 0)
            h.wait()
            _out_dma(t, 0, sout0).start()

        @pl.when(t % 2 == 1)
        def _():
            @pl.when(t >= 2)
            def _():
                _out_dma(t - 2, 1, sout1).wait()
            h = _fire_in(t, 1, sin1)
            lax.fori_loop(t * GPC, (t + 1) * GPC, _claim_body, 0)
            h.wait()
            _out_dma(t, 1, sout1).start()

        return 0

    lax.fori_loop(0, NCHUNK, _chunk_body, 0)
    _out_dma(NCHUNK - 2, 0, sout0).wait()
    _out_dma(NCHUNK - 1, 1, sout1).wait()

    def _compact_body(i, off):
        c = claim_v[pl.ds(i * 16, 16)]
        m = c >= 0
        pos = off + jnp.cumsum(m.astype(jnp.int32)) - 1
        plsc.store_scatter(wloc_v, [pos], lane + i * 16, mask=m)
        plsc.store_scatter(wb_v, [pos], c, mask=m)
        return off + plsc.all_reduce_population_count(m)[0]

    k_cnt = lax.fori_loop(0, CPAD // 16, _compact_body, jnp.int32(0))

    safe = jnp.maximum(k_cnt - 1, 0)
    rm = lax.rem(safe, 16)
    roff = safe - rm
    rms = jnp.broadcast_to(rm, (16,))
    lastloc = _vgather(wloc_v[pl.ds(roff, 16)], rms)
    lastb = _vgather(wb_v[pl.ds(roff, 16)], rms)
    for j in range(ACH // 16):
        plsc.store_scatter(wloc_v, [k_cnt + lane + j * 16], lastloc)
        plsc.store_scatter(wb_v, [k_cnt + lane + j * 16], lastb)

    pltpu.sync_copy(wloc_v, wloc_hbm.at[wid])
    pltpu.sync_copy(wb_v, wb_hbm.at[wid])
    kbuf_v[pl.ds(0, 16)] = jnp.broadcast_to(k_cnt, (16,))
    pltpu.sync_copy(kbuf_v, k_hbm.at[wid])


def _apply_body(out_hbm, nwr_hbm, s_hbm, wloc_hbm, wb_hbm, k_hbm,
                wloc_v, wb_v, kbuf_v, sbuf_v, buf_v, obuf_v,
                gs0, gs1, ss0, ss1):
    wid = lax.axis_index("c") * 16 + lax.axis_index("s")
    base = jnp.minimum(wid * RPW, NROWS - RPW)

    pltpu.sync_copy(wloc_hbm.at[wid], wloc_v)
    pltpu.sync_copy(wb_hbm.at[wid], wb_v)
    pltpu.sync_copy(k_hbm.at[wid], kbuf_v)
    k_cnt = kbuf_v[pl.ds(0, 16)][0]
    ntrips = (k_cnt + ACH - 1) // ACH

    def _fire_gathers(t, aslot, sem):
        for j in range(ACH // 16):
            bv = wb_v[pl.ds(t * ACH + j * 16, 16)]
            pltpu.async_copy(nwr_hbm.at[bv],
                             buf_v.at[aslot, pl.ds(j * 16, 16)], sem)
            pltpu.async_copy(s_hbm.at[bv],
                             sbuf_v.at[aslot, pl.ds(j * 16, 16)], sem)

    def _wait_gathers(t, aslot, sem):
        for j in range(ACH // 16):
            bv = wb_v[pl.ds(t * ACH + j * 16, 16)]
            pltpu.make_async_copy(
                nwr_hbm.at[bv], buf_v.at[aslot, pl.ds(j * 16, 16)],
                sem).wait()
            pltpu.make_async_copy(
                s_hbm.at[bv], sbuf_v.at[aslot, pl.ds(j * 16, 16)],
                sem).wait()

    def _scatter_dmas(t, aslot, sem):
        hs = []
        for j in range(ACH // 16):
            lv = wloc_v[pl.ds(t * ACH + j * 16, 16)] + base
            hs.append(pltpu.make_async_copy(
                obuf_v.at[aslot, pl.ds(j * 16, 16)], out_hbm.at[lv], sem))
        return hs

    @pl.when(ntrips > 0)
    def _():
        _fire_gathers(0, 0, gs0)

    def _trip(t, _):
        def _one(aslot, gsem, ssem):
            @pl.when(t + 1 < ntrips)
            def _():
                _fire_gathers(t + 1, 1 - aslot, gs1 if aslot == 0 else gs0)

            _wait_gathers(t, aslot, gsem)

            @pl.when(t >= 2)
            def _():
                for h in _scatter_dmas(t - 2, aslot, ssem):
                    h.wait()

            def _rowdiv(r, _):
                rr = lax.rem(r, 16)
                sval = _vgather(sbuf_v[aslot, pl.ds(r - rr, 16)],
                                jnp.broadcast_to(rr, (16,)))
                inv = 1.0 / sval
                for c in range(NCLS // 16):
                    obuf_v[aslot, r, pl.ds(c * 16, 16)] = (
                        buf_v[aslot, r, pl.ds(c * 16, 16)] * inv)
                return 0

            lax.fori_loop(0, ACH, _rowdiv, 0)
            for h in _scatter_dmas(t, aslot, ssem):
                h.start()

        @pl.when(t % 2 == 0)
        def _():
            _one(0, gs0, ss0)

        @pl.when(t % 2 == 1)
        def _():
            _one(1, gs1, ss1)

        return 0

    lax.fori_loop(0, ntrips, _trip, 0)

    def _drain_scatters(sem):
        for j in range(ACH // 16):
            lv = wloc_v[pl.ds(j * 16, 16)] + base
            pltpu.make_async_copy(obuf_v.at[0, pl.ds(j * 16, 16)],
                                  out_hbm.at[lv], sem).wait()

    @pl.when(ntrips >= 2)
    def _():
        _drain_scatters(ss0)
        _drain_scatters(ss1)

    @pl.when(ntrips == 1)
    def _():
        _drain_scatters(ss0)


def _sc_claims(weights, indices):
    mesh = plsc.VectorSubcoreMesh(core_axis_name="c", subcore_axis_name="s")
    f = functools.partial(
        pl.kernel,
        mesh=mesh,
        compiler_params=pltpu.CompilerParams(needs_layout_passes=False),
        out_type=[
            jax.ShapeDtypeStruct((NROWS, NCLS), jnp.float32),
            jax.ShapeDtypeStruct((NWORK, WLPAD), jnp.int32),
            jax.ShapeDtypeStruct((NWORK, WLPAD), jnp.int32),
            jax.ShapeDtypeStruct((NWORK, 16), jnp.int32),
        ],
        scratch_types=[
            pltpu.VMEM((IDXPAD,), jnp.int32),
            pltpu.VMEM((CPAD,), jnp.int32),
            pltpu.VMEM((WLPAD,), jnp.int32),
            pltpu.VMEM((WLPAD,), jnp.int32),
            pltpu.VMEM((16,), jnp.int32),
            pltpu.VMEM((2, CROWS, NCLS), jnp.float32),
            pltpu.SemaphoreType.DMA,
            pltpu.SemaphoreType.DMA,
            pltpu.SemaphoreType.DMA,
            pltpu.SemaphoreType.DMA,
        ],
    )(_claims_body)
    return f(weights, indices)


def _sc_apply(out_ref, nw_raw, s, wloc_a, wb_a, k_a):
    mesh = plsc.VectorSubcoreMesh(core_axis_name="c", subcore_axis_name="s")
    f = functools.partial(
        pl.kernel,
        mesh=mesh,
        compiler_params=pltpu.CompilerParams(needs_layout_passes=False),
        out_type=(),
        scratch_types=[
            pltpu.VMEM((WLPAD,), jnp.int32),
            pltpu.VMEM((WLPAD,), jnp.int32),
            pltpu.VMEM((16,), jnp.int32),
            pltpu.VMEM((2, ACH), jnp.float32),
            pltpu.VMEM((2, ACH, NCLS), jnp.float32),
            pltpu.VMEM((2, ACH, NCLS), jnp.float32),
            pltpu.SemaphoreType.DMA,
            pltpu.SemaphoreType.DMA,
            pltpu.SemaphoreType.DMA,
            pltpu.SemaphoreType.DMA,
        ],
    )(_apply_body)
    f(out_ref, nw_raw, s, wloc_a, wb_a, k_a)


def kernel(output, targets, weights, weak_labels, indices):
    updated0, wloc_a, wb_a, k_a = _sc_claims(weights, indices)
    w_g = jnp.take(weights, indices, axis=0)
    wl_g = jnp.take(weak_labels, indices, axis=0)
    grid = BATCH // BBLK
    row_spec = pl.BlockSpec((BBLK, NCLS), lambda i: (i, 0))
    nw_raw, lmat = pl.pallas_call(
        _dense_body,
        grid=(grid,),
        in_specs=[row_spec, row_spec, row_spec, row_spec],
        out_specs=[row_spec, pl.BlockSpec((1, 1), lambda i: (0, 0))],
        out_shape=[
            jax.ShapeDtypeStruct((BATCH, NCLS), jnp.float32),
            jax.ShapeDtypeStruct((1, 1), jnp.float32),
        ],
    )(output, targets, w_g, wl_g)
    s = jnp.sum(nw_raw, axis=1)
    uref = jax.new_ref(updated0)
    _sc_apply(uref, nw_raw, s, wloc_a, wb_a, k_a)
    updated = jax.freeze(uref)
    return lmat[0, 0], updated
